# Initial kernel scaffold; baseline (speedup 1.0000x reference)
import jax
import jax.numpy as jnp
from jax.experimental import pallas as pl


def kernel(node_feat, edge_feat, three_basis, three_cutoff, edge_dst, lg_src, lg_dst, segment_ids, W_atom, b_atom, W_bond, b_bond, W_gate, b_gate):
    raise NotImplementedError("write your pallas kernel here")


# speedup vs baseline: 29.0626x; 29.0626x over previous
import functools

import jax
import jax.numpy as jnp
from jax import lax
from jax.experimental import pallas as pl
from jax.experimental.pallas import tpu as pltpu
from jax.experimental.pallas import tpu_sc as plsc

N_NODES_C = 10000
N_EDGES_C = 160000
N_TRIPLES_C = 1280000
D_NODE_C = 128
D_EDGE_C = 64
NB_C = 16

NC = 2
NS = 16

CH = 512
KTOT = N_TRIPLES_C // CH
SEGS_PER_SC = N_EDGES_C // NC
ACC_ROWS = SEGS_PER_SC + 128
TRASH = SEGS_PER_SC


def _atoms_body(x_ref, w_ref, b_ref, o_ref):
    x = x_ref[...]
    o_ref[...] = jax.nn.sigmoid(
        jnp.dot(x, w_ref[...], preferred_element_type=jnp.float32) + b_ref[...]
    )


def _atoms_tc(node_feat, W_atom, b_atom):
    BN = 2000
    grid = (N_NODES_C // BN,)
    return pl.pallas_call(
        _atoms_body,
        grid=grid,
        in_specs=[
            pl.BlockSpec((BN, D_NODE_C), lambda i: (i, 0)),
            pl.BlockSpec((D_NODE_C, NB_C), lambda i: (0, 0)),
            pl.BlockSpec((1, NB_C), lambda i: (0, 0)),
        ],
        out_specs=pl.BlockSpec((BN, NB_C), lambda i: (i, 0)),
        out_shape=jax.ShapeDtypeStruct((N_NODES_C, NB_C), jnp.float32),
    )(node_feat, W_atom, b_atom.reshape(1, NB_C))


_B_CHUNK = 125
_B_NCHUNK = N_EDGES_C // _B_CHUNK
_B_PER_W = _B_NCHUNK // (NC * NS)
_B_WAVE = 8


def _edge_atoms_body(atoms_hbm, ed2_hbm, out_hbm, idxv, rowsv, sem):
    c = lax.axis_index("c")
    s = lax.axis_index("s")
    w = s * NC + c
    pltpu.sync_copy(ed2_hbm.at[pl.ds(w * _B_PER_W, _B_PER_W)], idxv)
    for wave in range(_B_PER_W // _B_WAVE):
        handles = []
        for j in range(_B_WAVE):
            k = wave * _B_WAVE + j
            handles.append(
                pltpu.async_copy(
                    atoms_hbm.at[idxv.at[k]],
                    rowsv.at[pl.ds(j * _B_CHUNK, _B_CHUNK)],
                    sem,
                )
            )
        for h in handles:
            h.wait()
        base = w * (_B_PER_W * _B_CHUNK) + wave * (_B_WAVE * _B_CHUNK)
        pltpu.sync_copy(rowsv, out_hbm.at[pl.ds(base, _B_WAVE * _B_CHUNK)])


def _edge_atoms_sc(atoms, edge_dst):
    mesh = plsc.VectorSubcoreMesh(
        core_axis_name="c", subcore_axis_name="s", num_cores=NC, num_subcores=NS
    )
    ed2 = edge_dst.reshape(_B_NCHUNK, _B_CHUNK)
    kern = pl.kernel(
        _edge_atoms_body,
        out_type=jax.ShapeDtypeStruct((N_EDGES_C, NB_C), jnp.float32),
        mesh=mesh,
        compiler_params=pltpu.CompilerParams(use_tc_tiling_on_sc=False),
        scratch_types=[
            pltpu.VMEM((_B_PER_W, _B_CHUNK), jnp.int32),
            pltpu.VMEM((_B_WAVE * _B_CHUNK, NB_C), jnp.float32),
            pltpu.SemaphoreType.DMA,
        ],
    )
    return kern(atoms, ed2)


def _triples_body(tb_hbm, seg2_hbm, lg2_hbm, ea_hbm, split_hbm, out_hbm,
                  segv, lgv, tbv, rowsv, prodv, ridx, zbuf, spl_v,
                  acc, gsem):
    c = lax.axis_index("c")
    s = lax.axis_index("s")

    pltpu.sync_copy(split_hbm, spl_v)
    split = spl_v[...][0]

    zrow = jnp.zeros((NB_C,), jnp.float32)

    def zbody(i, carry):
        zbuf[i, :] = zrow
        return carry

    lax.fori_loop(0, 313, zbody, 0)
    for q in range(16):
        pltpu.sync_copy(zbuf, acc.at[pl.ds(s * 5008 + q * 313, 313)])
    plsc.subcore_barrier()

    sc_lo = jnp.where(c == 0, 0, split)
    sc_hi = jnp.where(c == 0, split, N_TRIPLES_C)
    kc_lo = sc_lo // CH
    kc_hi = (sc_hi + (CH - 1)) // CH
    n = kc_hi - kc_lo
    ka = kc_lo + (n * s) // NS
    kb = kc_lo + (n * (s + 1)) // NS
    seg_base = c * SEGS_PER_SC

    def chunk_body(k, carry):
        st = k * CH
        pltpu.sync_copy(seg2_hbm.at[pl.ds(k * (CH // 16), CH // 16)], segv)
        pltpu.sync_copy(lg2_hbm.at[pl.ds(k * (CH // 128), CH // 128)], lgv)
        pltpu.sync_copy(tb_hbm.at[pl.ds(st, CH)], tbv)
        handles = []
        for j in range(CH // 128):
            handles.append(
                pltpu.async_copy(
                    ea_hbm.at[lgv.at[j]],
                    rowsv.at[pl.ds(j * 128, 128)],
                    gsem,
                )
            )
        for r in range(CH // 16):
            g = lax.iota(jnp.int32, 16) + (st + 16 * r)
            valid = (g >= sc_lo) & (g < sc_hi)
            rel = jnp.where(valid, segv[r, :] - seg_base, TRASH)
            ridx[r // 8, pl.ds(16 * (r % 8), 16)] = rel
        for h in handles:
            h.wait()

        def mul_body(i, carry2):
            for u in range(16):
                t = i * 16 + u
                prodv[t, :] = tbv[t, :] * rowsv[t, :]
            return carry2

        lax.fori_loop(0, CH // 16, mul_body, 0)
        for j in range(CH // 128):
            pltpu.sync_copy(
                prodv.at[pl.ds(j * 128, 128)],
                acc.at[ridx.at[j]],
                add=True,
            )
        return carry

    lax.fori_loop(ka, kb, chunk_body, 0)
    plsc.subcore_barrier()

    out_base = c * SEGS_PER_SC + s * (SEGS_PER_SC // NS)
    pltpu.sync_copy(
        acc.at[pl.ds(s * (SEGS_PER_SC // NS), SEGS_PER_SC // NS)],
        out_hbm.at[pl.ds(out_base, SEGS_PER_SC // NS)],
    )


def _triples_sc(three_basis, segment_ids, lg_dst, edge_atoms, split_arr):
    mesh = plsc.VectorSubcoreMesh(
        core_axis_name="c", subcore_axis_name="s", num_cores=NC, num_subcores=NS
    )
    seg2 = segment_ids.reshape(N_TRIPLES_C // 16, 16)
    lg2 = lg_dst.reshape(N_TRIPLES_C // 128, 128)
    kern = pl.kernel(
        _triples_body,
        out_type=jax.ShapeDtypeStruct((N_EDGES_C, NB_C), jnp.float32),
        mesh=mesh,
        compiler_params=pltpu.CompilerParams(use_tc_tiling_on_sc=False),
        scratch_types=[
            pltpu.VMEM((CH // 16, 16), jnp.int32),
            pltpu.VMEM((CH // 128, 128), jnp.int32),
            pltpu.VMEM((CH, NB_C), jnp.float32),
            pltpu.VMEM((CH, NB_C), jnp.float32),
            pltpu.VMEM((CH, NB_C), jnp.float32),
            pltpu.VMEM((CH // 128, 128), jnp.int32),
            pltpu.VMEM((313, NB_C), jnp.float32),
            pltpu.VMEM((16,), jnp.int32),
            pltpu.VMEM_SHARED((ACC_ROWS, NB_C), jnp.float32),
            pltpu.SemaphoreType.DMA,
        ],
    )
    return kern(three_basis, seg2, lg2, edge_atoms, split_arr)


def _mlp_body(nb_ref, ef_ref, wb_ref, bb_ref, wg_ref, bg_ref, o_ref):
    nb = nb_ref[...]
    u = jnp.dot(nb, wb_ref[...], preferred_element_type=jnp.float32) + bb_ref[...]
    g = jax.nn.sigmoid(
        jnp.dot(nb, wg_ref[...], preferred_element_type=jnp.float32) + bg_ref[...]
    )
    o_ref[...] = ef_ref[...] + u * g


def _mlp_tc(new_bonds, edge_feat, W_bond, b_bond, W_gate, b_gate):
    BE = 2000
    grid = (N_EDGES_C // BE,)
    return pl.pallas_call(
        _mlp_body,
        grid=grid,
        in_specs=[
            pl.BlockSpec((BE, NB_C), lambda i: (i, 0)),
            pl.BlockSpec((BE, D_EDGE_C), lambda i: (i, 0)),
            pl.BlockSpec((NB_C, D_EDGE_C), lambda i: (0, 0)),
            pl.BlockSpec((1, D_EDGE_C), lambda i: (0, 0)),
            pl.BlockSpec((NB_C, D_EDGE_C), lambda i: (0, 0)),
            pl.BlockSpec((1, D_EDGE_C), lambda i: (0, 0)),
        ],
        out_specs=pl.BlockSpec((BE, D_EDGE_C), lambda i: (i, 0)),
        out_shape=jax.ShapeDtypeStruct((N_EDGES_C, D_EDGE_C), jnp.float32),
    )(new_bonds, edge_feat, W_bond, b_bond.reshape(1, D_EDGE_C),
      W_gate, b_gate.reshape(1, D_EDGE_C))


def kernel(node_feat, edge_feat, three_basis, three_cutoff, edge_dst, lg_src,
           lg_dst, segment_ids, W_atom, b_atom, W_bond, b_bond, W_gate, b_gate):
    del three_cutoff, lg_src
    edge_dst = edge_dst.astype(jnp.int32)
    lg_dst = lg_dst.astype(jnp.int32)
    segment_ids = segment_ids.astype(jnp.int32)

    atoms = _atoms_tc(node_feat, W_atom, b_atom)
    edge_atoms = _edge_atoms_sc(atoms, edge_dst)

    split = jnp.searchsorted(
        segment_ids, jnp.int32(SEGS_PER_SC), side="left"
    ).astype(jnp.int32)
    split_arr = jnp.full((16,), split, dtype=jnp.int32)

    new_bonds = _triples_sc(three_basis, segment_ids, lg_dst, edge_atoms, split_arr)
    return _mlp_tc(new_bonds, edge_feat, W_bond, b_bond, W_gate, b_gate)


# speedup vs baseline: 29.1154x; 1.0018x over previous
import functools

import jax
import jax.numpy as jnp
from jax import lax
from jax.experimental import pallas as pl
from jax.experimental.pallas import tpu as pltpu
from jax.experimental.pallas import tpu_sc as plsc

N_NODES_C = 10000
N_EDGES_C = 160000
N_TRIPLES_C = 1280000
D_NODE_C = 128
D_EDGE_C = 64
NB_C = 16

NC = 2
NS = 16

CH = 512
KTOT = N_TRIPLES_C // CH
SEGS_PER_SC = N_EDGES_C // NC
ACC_ROWS = SEGS_PER_SC + 128
TRASH = SEGS_PER_SC


def _atoms_body(x_ref, w_ref, b_ref, o_ref):
    x = x_ref[...]
    o_ref[...] = jax.nn.sigmoid(
        jnp.dot(x, w_ref[...], preferred_element_type=jnp.float32) + b_ref[...]
    )


def _atoms_tc(node_feat, W_atom, b_atom):
    BN = 2000
    grid = (N_NODES_C // BN,)
    return pl.pallas_call(
        _atoms_body,
        grid=grid,
        in_specs=[
            pl.BlockSpec((BN, D_NODE_C), lambda i: (i, 0)),
            pl.BlockSpec((D_NODE_C, NB_C), lambda i: (0, 0)),
            pl.BlockSpec((1, NB_C), lambda i: (0, 0)),
        ],
        out_specs=pl.BlockSpec((BN, NB_C), lambda i: (i, 0)),
        out_shape=jax.ShapeDtypeStruct((N_NODES_C, NB_C), jnp.float32),
    )(node_feat, W_atom, b_atom.reshape(1, NB_C))


_B_CHUNK = 128
_B_PER_W = 39
_B_EDGES_W = _B_CHUNK * _B_PER_W
_B_TAIL = N_EDGES_C - 32 * _B_EDGES_W


def _edge_atoms_body(atoms_hbm, ed_hbm, out_hbm, idxv, rowsv, sem):
    c = lax.axis_index("c")
    s = lax.axis_index("s")
    w = s * NC + c
    base_e = w * _B_EDGES_W
    pltpu.sync_copy(ed_hbm.at[pl.ds(base_e, _B_EDGES_W)], idxv)
    for wave in range(5):
        nj = 8 if wave < 4 else 7
        handles = []
        for j in range(nj):
            k = wave * 8 + j
            handles.append(
                pltpu.async_copy(
                    atoms_hbm.at[idxv.at[pl.ds(k * _B_CHUNK, _B_CHUNK)]],
                    rowsv.at[pl.ds(j * _B_CHUNK, _B_CHUNK)],
                    sem,
                )
            )
        for h in handles:
            h.wait()
        pltpu.sync_copy(
            rowsv.at[pl.ds(0, nj * _B_CHUNK)],
            out_hbm.at[pl.ds(base_e + wave * 8 * _B_CHUNK, nj * _B_CHUNK)],
        )

    @pl.when(w < 2)
    def _tail():
        tb = 32 * _B_EDGES_W + w * _B_CHUNK
        pltpu.sync_copy(ed_hbm.at[pl.ds(tb, _B_CHUNK)], idxv.at[pl.ds(0, _B_CHUNK)])
        pltpu.async_copy(
            atoms_hbm.at[idxv.at[pl.ds(0, _B_CHUNK)]],
            rowsv.at[pl.ds(0, _B_CHUNK)],
            sem,
        ).wait()
        pltpu.sync_copy(rowsv.at[pl.ds(0, _B_CHUNK)], out_hbm.at[pl.ds(tb, _B_CHUNK)])


def _edge_atoms_sc(atoms, edge_dst):
    mesh = plsc.VectorSubcoreMesh(
        core_axis_name="c", subcore_axis_name="s", num_cores=NC, num_subcores=NS
    )
    kern = pl.kernel(
        _edge_atoms_body,
        out_type=jax.ShapeDtypeStruct((N_EDGES_C, NB_C), jnp.float32),
        mesh=mesh,
        compiler_params=pltpu.CompilerParams(use_tc_tiling_on_sc=False),
        scratch_types=[
            pltpu.VMEM((_B_EDGES_W,), jnp.int32),
            pltpu.VMEM((8 * _B_CHUNK, NB_C), jnp.float32),
            pltpu.SemaphoreType.DMA,
        ],
    )
    return kern(atoms, edge_dst)


def _triples_body(tb_hbm, seg_hbm, lg_hbm, ea_hbm, split_hbm, out_hbm,
                  segv, lgv, tbv, rowsv, ridx, zbuf, spl_v,
                  acc, gsem):
    c = lax.axis_index("c")
    s = lax.axis_index("s")

    pltpu.sync_copy(split_hbm, spl_v)
    split = spl_v[...][0]

    zrow = jnp.zeros((NB_C,), jnp.float32)

    def zbody(i, carry):
        zbuf[i, :] = zrow
        return carry

    lax.fori_loop(0, 313, zbody, 0)
    for q in range(16):
        pltpu.sync_copy(zbuf, acc.at[pl.ds(s * 5008 + q * 313, 313)])
    plsc.subcore_barrier()

    sc_lo = jnp.where(c == 0, 0, split)
    sc_hi = jnp.where(c == 0, split, N_TRIPLES_C)
    kc_lo = sc_lo // CH
    kc_hi = (sc_hi + (CH - 1)) // CH
    n = kc_hi - kc_lo
    ka = kc_lo + (n * s) // NS
    kb = kc_lo + (n * (s + 1)) // NS
    seg_base = c * SEGS_PER_SC

    def chunk_body(k, carry):
        st = k * CH
        pltpu.sync_copy(seg_hbm.at[pl.ds(st, CH)], segv)
        pltpu.sync_copy(lg_hbm.at[pl.ds(st, CH)], lgv)
        pltpu.sync_copy(tb_hbm.at[pl.ds(st, CH)], tbv)
        handles = []
        for j in range(CH // 128):
            handles.append(
                pltpu.async_copy(
                    ea_hbm.at[lgv.at[pl.ds(j * 128, 128)]],
                    rowsv.at[pl.ds(j * 128, 128)],
                    gsem,
                )
            )
        for r in range(CH // 16):
            g = lax.iota(jnp.int32, 16) + (st + 16 * r)
            valid = (g >= sc_lo) & (g < sc_hi)
            rel = jnp.where(valid, segv[pl.ds(16 * r, 16)] - seg_base, TRASH)
            ridx[r // 8, pl.ds(16 * (r % 8), 16)] = rel
        for h in handles:
            h.wait()

        def mul_body(i, carry2):
            for u in range(16):
                t = i * 16 + u
                rowsv[t, :] = tbv[t, :] * rowsv[t, :]
            return carry2

        lax.fori_loop(0, CH // 16, mul_body, 0)
        for j in range(CH // 128):
            pltpu.sync_copy(
                rowsv.at[pl.ds(j * 128, 128)],
                acc.at[ridx.at[j]],
                add=True,
            )
        return carry

    lax.fori_loop(ka, kb, chunk_body, 0)
    plsc.subcore_barrier()

    out_base = c * SEGS_PER_SC + s * (SEGS_PER_SC // NS)
    pltpu.sync_copy(
        acc.at[pl.ds(s * (SEGS_PER_SC // NS), SEGS_PER_SC // NS)],
        out_hbm.at[pl.ds(out_base, SEGS_PER_SC // NS)],
    )


def _triples_sc(three_basis, segment_ids, lg_dst, edge_atoms, split_arr):
    mesh = plsc.VectorSubcoreMesh(
        core_axis_name="c", subcore_axis_name="s", num_cores=NC, num_subcores=NS
    )
    kern = pl.kernel(
        _triples_body,
        out_type=jax.ShapeDtypeStruct((N_EDGES_C, NB_C), jnp.float32),
        mesh=mesh,
        compiler_params=pltpu.CompilerParams(use_tc_tiling_on_sc=False),
        scratch_types=[
            pltpu.VMEM((CH,), jnp.int32),
            pltpu.VMEM((CH,), jnp.int32),
            pltpu.VMEM((CH, NB_C), jnp.float32),
            pltpu.VMEM((CH, NB_C), jnp.float32),
            pltpu.VMEM((CH // 128, 128), jnp.int32),
            pltpu.VMEM((313, NB_C), jnp.float32),
            pltpu.VMEM((16,), jnp.int32),
            pltpu.VMEM_SHARED((ACC_ROWS, NB_C), jnp.float32),
            pltpu.SemaphoreType.DMA,
        ],
    )
    return kern(three_basis, segment_ids, lg_dst, edge_atoms, split_arr)


def _mlp_body(nb_ref, ef_ref, wb_ref, bb_ref, wg_ref, bg_ref, o_ref):
    nb = nb_ref[...]
    u = jnp.dot(nb, wb_ref[...], preferred_element_type=jnp.float32) + bb_ref[...]
    g = jax.nn.sigmoid(
        jnp.dot(nb, wg_ref[...], preferred_element_type=jnp.float32) + bg_ref[...]
    )
    o_ref[...] = ef_ref[...] + u * g


def _mlp_tc(new_bonds, edge_feat, W_bond, b_bond, W_gate, b_gate):
    BE = 2000
    grid = (N_EDGES_C // BE,)
    return pl.pallas_call(
        _mlp_body,
        grid=grid,
        in_specs=[
            pl.BlockSpec((BE, NB_C), lambda i: (i, 0)),
            pl.BlockSpec((BE, D_EDGE_C), lambda i: (i, 0)),
            pl.BlockSpec((NB_C, D_EDGE_C), lambda i: (0, 0)),
            pl.BlockSpec((1, D_EDGE_C), lambda i: (0, 0)),
            pl.BlockSpec((NB_C, D_EDGE_C), lambda i: (0, 0)),
            pl.BlockSpec((1, D_EDGE_C), lambda i: (0, 0)),
        ],
        out_specs=pl.BlockSpec((BE, D_EDGE_C), lambda i: (i, 0)),
        out_shape=jax.ShapeDtypeStruct((N_EDGES_C, D_EDGE_C), jnp.float32),
    )(new_bonds, edge_feat, W_bond, b_bond.reshape(1, D_EDGE_C),
      W_gate, b_gate.reshape(1, D_EDGE_C))


def kernel(node_feat, edge_feat, three_basis, three_cutoff, edge_dst, lg_src,
           lg_dst, segment_ids, W_atom, b_atom, W_bond, b_bond, W_gate, b_gate):
    del three_cutoff, lg_src
    edge_dst = edge_dst.astype(jnp.int32)
    lg_dst = lg_dst.astype(jnp.int32)
    segment_ids = segment_ids.astype(jnp.int32)

    atoms = _atoms_tc(node_feat, W_atom, b_atom)
    edge_atoms = _edge_atoms_sc(atoms, edge_dst)

    split = jnp.sum(
        (segment_ids < SEGS_PER_SC).astype(jnp.int32), dtype=jnp.int32
    )
    split_arr = jnp.full((16,), split, dtype=jnp.int32)

    new_bonds = _triples_sc(three_basis, segment_ids, lg_dst, edge_atoms, split_arr)
    return _mlp_tc(new_bonds, edge_feat, W_bond, b_bond, W_gate, b_gate)


# speedup vs baseline: 34.7207x; 1.1925x over previous
import functools

import jax
import jax.numpy as jnp
from jax import lax
from jax.experimental import pallas as pl
from jax.experimental.pallas import tpu as pltpu
from jax.experimental.pallas import tpu_sc as plsc

N_NODES_C = 10000
N_EDGES_C = 160000
N_TRIPLES_C = 1280000
D_NODE_C = 128
D_EDGE_C = 64
NB_C = 16

NC = 2
NS = 16

CH = 512
KTOT = N_TRIPLES_C // CH
SEGS_PER_SC = N_EDGES_C // NC
ACC_ROWS = SEGS_PER_SC + 128
TRASH = SEGS_PER_SC
NJ = CH // 128


def _atoms_body(x_ref, w_ref, b_ref, o_ref):
    x = x_ref[...]
    o_ref[...] = jax.nn.sigmoid(
        jnp.dot(x, w_ref[...], preferred_element_type=jnp.float32) + b_ref[...]
    )


def _atoms_tc(node_feat, W_atom, b_atom):
    BN = 2000
    grid = (N_NODES_C // BN,)
    return pl.pallas_call(
        _atoms_body,
        grid=grid,
        in_specs=[
            pl.BlockSpec((BN, D_NODE_C), lambda i: (i, 0)),
            pl.BlockSpec((D_NODE_C, NB_C), lambda i: (0, 0)),
            pl.BlockSpec((1, NB_C), lambda i: (0, 0)),
        ],
        out_specs=pl.BlockSpec((BN, NB_C), lambda i: (i, 0)),
        out_shape=jax.ShapeDtypeStruct((N_NODES_C, NB_C), jnp.float32),
    )(node_feat, W_atom, b_atom.reshape(1, NB_C))


_B_CHUNK = 128
_B_PER_W = 39
_B_EDGES_W = _B_CHUNK * _B_PER_W
_B_TAIL = N_EDGES_C - 32 * _B_EDGES_W


def _edge_atoms_body(atoms_hbm, ed_hbm, out_hbm, idxv, rowsv, sem):
    c = lax.axis_index("c")
    s = lax.axis_index("s")
    w = s * NC + c
    base_e = w * _B_EDGES_W
    pltpu.sync_copy(ed_hbm.at[pl.ds(base_e, _B_EDGES_W)], idxv)
    for wave in range(5):
        nj = 8 if wave < 4 else 7
        handles = []
        for j in range(nj):
            k = wave * 8 + j
            handles.append(
                pltpu.async_copy(
                    atoms_hbm.at[idxv.at[pl.ds(k * _B_CHUNK, _B_CHUNK)]],
                    rowsv.at[pl.ds(j * _B_CHUNK, _B_CHUNK)],
                    sem,
                )
            )
        for h in handles:
            h.wait()
        pltpu.sync_copy(
            rowsv.at[pl.ds(0, nj * _B_CHUNK)],
            out_hbm.at[pl.ds(base_e + wave * 8 * _B_CHUNK, nj * _B_CHUNK)],
        )

    @pl.when(w < 2)
    def _tail():
        tb = 32 * _B_EDGES_W + w * _B_CHUNK
        pltpu.sync_copy(ed_hbm.at[pl.ds(tb, _B_CHUNK)], idxv.at[pl.ds(0, _B_CHUNK)])
        pltpu.async_copy(
            atoms_hbm.at[idxv.at[pl.ds(0, _B_CHUNK)]],
            rowsv.at[pl.ds(0, _B_CHUNK)],
            sem,
        ).wait()
        pltpu.sync_copy(rowsv.at[pl.ds(0, _B_CHUNK)], out_hbm.at[pl.ds(tb, _B_CHUNK)])


def _edge_atoms_sc(atoms, edge_dst):
    mesh = plsc.VectorSubcoreMesh(
        core_axis_name="c", subcore_axis_name="s", num_cores=NC, num_subcores=NS
    )
    kern = pl.kernel(
        _edge_atoms_body,
        out_type=jax.ShapeDtypeStruct((N_EDGES_C, NB_C), jnp.float32),
        mesh=mesh,
        compiler_params=pltpu.CompilerParams(use_tc_tiling_on_sc=False),
        scratch_types=[
            pltpu.VMEM((_B_EDGES_W,), jnp.int32),
            pltpu.VMEM((8 * _B_CHUNK, NB_C), jnp.float32),
            pltpu.SemaphoreType.DMA,
        ],
    )
    return kern(atoms, edge_dst)


def _triples_body(tb_hbm, seg_hbm, lg_hbm, ea_hbm, split_hbm, out_hbm,
                  segv, lgv, tbv, rowsv, ridx, zbuf, spl_v,
                  acc, lsem0, lsem1, gsem0, gsem1, ssem0, ssem1):
    c = lax.axis_index("c")
    s = lax.axis_index("s")
    lsem = (lsem0, lsem1)
    gsem = (gsem0, gsem1)
    ssem = (ssem0, ssem1)

    pltpu.sync_copy(split_hbm, spl_v)
    split = spl_v[...][0]

    zrow = jnp.zeros((NB_C,), jnp.float32)

    def zbody(i, carry):
        zbuf[i, :] = zrow
        return carry

    lax.fori_loop(0, 313, zbody, 0)
    for q in range(16):
        pltpu.sync_copy(zbuf, acc.at[pl.ds(s * 5008 + q * 313, 313)])
    plsc.subcore_barrier()

    sc_lo = jnp.where(c == 0, 0, split)
    sc_hi = jnp.where(c == 0, split, N_TRIPLES_C)
    kc_lo = sc_lo // CH
    kc_hi = (sc_hi + (CH - 1)) // CH
    n = kc_hi - kc_lo
    ka = kc_lo + (n * s) // NS
    kb = kc_lo + (n * (s + 1)) // NS
    nch = kb - ka
    seg_base = c * SEGS_PER_SC

    def issue_linear(k, b):
        st = k * CH
        pltpu.async_copy(seg_hbm.at[pl.ds(st, CH)], segv.at[b], lsem[b])
        pltpu.async_copy(lg_hbm.at[pl.ds(st, CH)], lgv.at[b], lsem[b])
        pltpu.async_copy(tb_hbm.at[pl.ds(st * NB_C, CH * NB_C)], tbv.at[b], lsem[b])

    def wait_linear(b):
        pltpu.make_async_copy(seg_hbm.at[pl.ds(0, CH)], segv.at[b], lsem[b]).wait()
        pltpu.make_async_copy(lg_hbm.at[pl.ds(0, CH)], lgv.at[b], lsem[b]).wait()
        pltpu.make_async_copy(tb_hbm.at[pl.ds(0, CH * NB_C)], tbv.at[b], lsem[b]).wait()

    def issue_gathers(b):
        hs = []
        for j in range(NJ):
            hs.append(pltpu.async_copy(
                ea_hbm.at[lgv.at[b, pl.ds(j * 128, 128)]],
                rowsv.at[b, pl.ds(j * 128, 128)],
                gsem[b],
            ))
        return hs

    def issue_scatters(b):
        for j in range(NJ):
            pltpu.async_copy(
                rowsv.at[b, pl.ds(j * 128, 128)],
                acc.at[ridx.at[b, j]],
                ssem[b],
                add=True,
            )

    def drain_scatters(b):
        for j in range(NJ):
            pltpu.make_async_copy(
                rowsv.at[b, pl.ds(j * 128, 128)],
                acc.at[ridx.at[b, j]],
                ssem[b],
            ).wait()

    def compute_ridx(k, b):
        st = k * CH
        for r in range(CH // 16):
            g = lax.iota(jnp.int32, 16) + (st + 16 * r)
            valid = (g >= sc_lo) & (g < sc_hi)
            rel = jnp.where(valid, segv[b, pl.ds(16 * r, 16)] - seg_base, TRASH)
            ridx[b, r // 8, pl.ds(16 * (r % 8), 16)] = rel

    def mul(b):
        def mul_body(i, carry2):
            for u in range(16):
                t = i * 16 + u
                rowsv[b, t, :] = tbv[b, pl.ds(t * NB_C, NB_C)] * rowsv[b, t, :]
            return carry2

        lax.fori_loop(0, CH // 16, mul_body, 0)

    trash_row = jnp.full((16,), TRASH, jnp.int32)

    @pl.when(nch > 0)
    def _run():
        for b in range(2):
            for j in range(NJ):
                for r in range(8):
                    ridx[b, j, pl.ds(16 * r, 16)] = trash_row
        issue_scatters(0)
        issue_scatters(1)
        issue_linear(ka, 0)

        npairs = (nch + 1) // 2

        def pair_body(i2, carry):
            kA = ka + 2 * i2
            kB = kA + 1
            vB = kB < kb
            wait_linear(0)
            drain_scatters(0)
            hsA = issue_gathers(0)

            @pl.when(vB)
            def _():
                issue_linear(kB, 1)

            compute_ridx(kA, 0)
            for h in hsA:
                h.wait()
            mul(0)
            issue_scatters(0)

            @pl.when(kA + 2 < kb)
            def _():
                issue_linear(kA + 2, 0)

            @pl.when(vB)
            def _():
                wait_linear(1)
                drain_scatters(1)
                hsB = issue_gathers(1)
                compute_ridx(kB, 1)
                for h in hsB:
                    h.wait()
                mul(1)
                issue_scatters(1)

            return carry

        lax.fori_loop(0, npairs, pair_body, 0)
        drain_scatters(0)
        drain_scatters(1)

    plsc.subcore_barrier()

    out_base = c * SEGS_PER_SC + s * (SEGS_PER_SC // NS)
    pltpu.sync_copy(
        acc.at[pl.ds(s * (SEGS_PER_SC // NS), SEGS_PER_SC // NS)],
        out_hbm.at[pl.ds(out_base, SEGS_PER_SC // NS)],
    )


def _triples_sc(tb_flat, segment_ids, lg_dst, edge_atoms, split_arr):
    mesh = plsc.VectorSubcoreMesh(
        core_axis_name="c", subcore_axis_name="s", num_cores=NC, num_subcores=NS
    )
    kern = pl.kernel(
        _triples_body,
        out_type=jax.ShapeDtypeStruct((N_EDGES_C, NB_C), jnp.float32),
        mesh=mesh,
        compiler_params=pltpu.CompilerParams(use_tc_tiling_on_sc=False),
        scratch_types=[
            pltpu.VMEM((2, CH), jnp.int32),
            pltpu.VMEM((2, CH), jnp.int32),
            pltpu.VMEM((2, CH * NB_C), jnp.float32),
            pltpu.VMEM((2, CH, NB_C), jnp.float32),
            pltpu.VMEM((2, NJ, 128), jnp.int32),
            pltpu.VMEM((313, NB_C), jnp.float32),
            pltpu.VMEM((16,), jnp.int32),
            pltpu.VMEM_SHARED((ACC_ROWS, NB_C), jnp.float32),
            pltpu.SemaphoreType.DMA,
            pltpu.SemaphoreType.DMA,
            pltpu.SemaphoreType.DMA,
            pltpu.SemaphoreType.DMA,
            pltpu.SemaphoreType.DMA,
            pltpu.SemaphoreType.DMA,
        ],
    )
    return kern(tb_flat, segment_ids, lg_dst, edge_atoms, split_arr)


def _mlp_body(nb_ref, ef_ref, wb_ref, bb_ref, wg_ref, bg_ref, o_ref):
    nb = nb_ref[...]
    u = jnp.dot(nb, wb_ref[...], preferred_element_type=jnp.float32) + bb_ref[...]
    g = jax.nn.sigmoid(
        jnp.dot(nb, wg_ref[...], preferred_element_type=jnp.float32) + bg_ref[...]
    )
    o_ref[...] = ef_ref[...] + u * g


def _mlp_tc(new_bonds, edge_feat, W_bond, b_bond, W_gate, b_gate):
    BE = 2000
    grid = (N_EDGES_C // BE,)
    return pl.pallas_call(
        _mlp_body,
        grid=grid,
        in_specs=[
            pl.BlockSpec((BE, NB_C), lambda i: (i, 0)),
            pl.BlockSpec((BE, D_EDGE_C), lambda i: (i, 0)),
            pl.BlockSpec((NB_C, D_EDGE_C), lambda i: (0, 0)),
            pl.BlockSpec((1, D_EDGE_C), lambda i: (0, 0)),
            pl.BlockSpec((NB_C, D_EDGE_C), lambda i: (0, 0)),
            pl.BlockSpec((1, D_EDGE_C), lambda i: (0, 0)),
        ],
        out_specs=pl.BlockSpec((BE, D_EDGE_C), lambda i: (i, 0)),
        out_shape=jax.ShapeDtypeStruct((N_EDGES_C, D_EDGE_C), jnp.float32),
    )(new_bonds, edge_feat, W_bond, b_bond.reshape(1, D_EDGE_C),
      W_gate, b_gate.reshape(1, D_EDGE_C))


def kernel(node_feat, edge_feat, three_basis, three_cutoff, edge_dst, lg_src,
           lg_dst, segment_ids, W_atom, b_atom, W_bond, b_bond, W_gate, b_gate):
    del three_cutoff, lg_src
    edge_dst = edge_dst.astype(jnp.int32)
    lg_dst = lg_dst.astype(jnp.int32)
    segment_ids = segment_ids.astype(jnp.int32)

    atoms = _atoms_tc(node_feat, W_atom, b_atom)
    edge_atoms = _edge_atoms_sc(atoms, edge_dst)

    split = jnp.sum(
        (segment_ids < SEGS_PER_SC).astype(jnp.int32), dtype=jnp.int32
    )
    split_arr = jnp.full((16,), split, dtype=jnp.int32)

    tb_flat = three_basis.reshape(-1)
    new_bonds = _triples_sc(tb_flat, segment_ids, lg_dst, edge_atoms, split_arr)
    return _mlp_tc(new_bonds, edge_feat, W_bond, b_bond, W_gate, b_gate)


# speedup vs baseline: 38.6990x; 1.1146x over previous
import functools

import jax
import jax.numpy as jnp
from jax import lax
from jax.experimental import pallas as pl
from jax.experimental.pallas import tpu as pltpu
from jax.experimental.pallas import tpu_sc as plsc

N_NODES_C = 10000
N_EDGES_C = 160000
N_TRIPLES_C = 1280000
D_NODE_C = 128
D_EDGE_C = 64
NB_C = 16

NC = 2
NS = 16

CH = 512
KTOT = N_TRIPLES_C // CH
SEGS_PER_SC = N_EDGES_C // NC
ACC_ROWS = SEGS_PER_SC + 128
TRASH = SEGS_PER_SC
NJ = CH // 128


def _atoms_body(x_ref, w_ref, b_ref, o_ref):
    x = x_ref[...]
    o_ref[...] = jax.nn.sigmoid(
        jnp.dot(x, w_ref[...], preferred_element_type=jnp.float32) + b_ref[...]
    )


def _atoms_tc(node_feat, W_atom, b_atom):
    BN = 2000
    grid = (N_NODES_C // BN,)
    return pl.pallas_call(
        _atoms_body,
        grid=grid,
        in_specs=[
            pl.BlockSpec((BN, D_NODE_C), lambda i: (i, 0)),
            pl.BlockSpec((D_NODE_C, NB_C), lambda i: (0, 0)),
            pl.BlockSpec((1, NB_C), lambda i: (0, 0)),
        ],
        out_specs=pl.BlockSpec((BN, NB_C), lambda i: (i, 0)),
        out_shape=jax.ShapeDtypeStruct((N_NODES_C, NB_C), jnp.float32),
    )(node_feat, W_atom, b_atom.reshape(1, NB_C))


_B_CHUNK = 128
_B_PER_W = 39
_B_EDGES_W = _B_CHUNK * _B_PER_W
_B_TAIL = N_EDGES_C - 32 * _B_EDGES_W


def _edge_atoms_body(atoms_hbm, ed_hbm, out_hbm, idxv, rowsv, sem):
    c = lax.axis_index("c")
    s = lax.axis_index("s")
    w = s * NC + c
    base_e = w * _B_EDGES_W
    pltpu.sync_copy(ed_hbm.at[pl.ds(base_e, _B_EDGES_W)], idxv)
    for wave in range(5):
        nj = 8 if wave < 4 else 7
        handles = []
        for j in range(nj):
            k = wave * 8 + j
            handles.append(
                pltpu.async_copy(
                    atoms_hbm.at[idxv.at[pl.ds(k * _B_CHUNK, _B_CHUNK)]],
                    rowsv.at[pl.ds(j * _B_CHUNK, _B_CHUNK)],
                    sem,
                )
            )
        for h in handles:
            h.wait()
        pltpu.sync_copy(
            rowsv.at[pl.ds(0, nj * _B_CHUNK)],
            out_hbm.at[pl.ds(base_e + wave * 8 * _B_CHUNK, nj * _B_CHUNK)],
        )

    @pl.when(w < 2)
    def _tail():
        tb = 32 * _B_EDGES_W + w * _B_CHUNK
        pltpu.sync_copy(ed_hbm.at[pl.ds(tb, _B_CHUNK)], idxv.at[pl.ds(0, _B_CHUNK)])
        pltpu.async_copy(
            atoms_hbm.at[idxv.at[pl.ds(0, _B_CHUNK)]],
            rowsv.at[pl.ds(0, _B_CHUNK)],
            sem,
        ).wait()
        pltpu.sync_copy(rowsv.at[pl.ds(0, _B_CHUNK)], out_hbm.at[pl.ds(tb, _B_CHUNK)])


def _edge_atoms_sc(atoms, edge_dst):
    mesh = plsc.VectorSubcoreMesh(
        core_axis_name="c", subcore_axis_name="s", num_cores=NC, num_subcores=NS
    )
    kern = pl.kernel(
        _edge_atoms_body,
        out_type=jax.ShapeDtypeStruct((N_EDGES_C, NB_C), jnp.float32),
        mesh=mesh,
        compiler_params=pltpu.CompilerParams(use_tc_tiling_on_sc=False),
        scratch_types=[
            pltpu.VMEM((_B_EDGES_W,), jnp.int32),
            pltpu.VMEM((8 * _B_CHUNK, NB_C), jnp.float32),
            pltpu.SemaphoreType.DMA,
        ],
    )
    return kern(atoms, edge_dst)


def _triples_body(tb_hbm, seg_hbm, lg_hbm, ea_hbm, split_hbm, out_hbm,
                  segv, lgv, tbv, rowsv, ridx, zbuf, spl_v,
                  acc, lsem0, lsem1, gsem0, gsem1, ssem0, ssem1):
    c = lax.axis_index("c")
    s = lax.axis_index("s")
    lsem = (lsem0, lsem1)
    gsem = (gsem0, gsem1)
    ssem = (ssem0, ssem1)

    pltpu.sync_copy(split_hbm, spl_v)
    split = spl_v[...][0]

    zrow = jnp.zeros((NB_C,), jnp.float32)

    def zbody(i, carry):
        zbuf[i, :] = zrow
        return carry

    lax.fori_loop(0, 313, zbody, 0)
    for q in range(16):
        pltpu.sync_copy(zbuf, acc.at[pl.ds(s * 5008 + q * 313, 313)])
    plsc.subcore_barrier()

    sc_lo = jnp.where(c == 0, 0, split)
    sc_hi = jnp.where(c == 0, split, N_TRIPLES_C)
    kc_lo = sc_lo // CH
    kc_hi = (sc_hi + (CH - 1)) // CH
    n = kc_hi - kc_lo
    ka = kc_lo + (n * s) // NS
    kb = kc_lo + (n * (s + 1)) // NS
    nch = kb - ka
    seg_base = c * SEGS_PER_SC

    def issue_linear(k, b):
        st = k * CH
        pltpu.async_copy(seg_hbm.at[pl.ds(st, CH)], segv.at[b], lsem[b])
        pltpu.async_copy(lg_hbm.at[pl.ds(st, CH)], lgv.at[b], lsem[b])
        pltpu.async_copy(tb_hbm.at[pl.ds(st * NB_C, CH * NB_C)], tbv.at[b], lsem[b])

    def wait_linear(b):
        pltpu.make_async_copy(seg_hbm.at[pl.ds(0, CH)], segv.at[b], lsem[b]).wait()
        pltpu.make_async_copy(lg_hbm.at[pl.ds(0, CH)], lgv.at[b], lsem[b]).wait()
        pltpu.make_async_copy(tb_hbm.at[pl.ds(0, CH * NB_C)], tbv.at[b], lsem[b]).wait()

    def issue_gathers(b):
        hs = []
        for j in range(NJ):
            hs.append(pltpu.async_copy(
                ea_hbm.at[lgv.at[b, pl.ds(j * 128, 128)]],
                rowsv.at[b, pl.ds(j * 128, 128)],
                gsem[b],
            ))
        return hs

    def issue_scatters(b):
        for j in range(NJ):
            pltpu.async_copy(
                rowsv.at[b, pl.ds(j * 128, 128)],
                acc.at[ridx.at[b, j]],
                ssem[b],
                add=True,
            )

    def drain_scatters(b):
        for j in range(NJ):
            pltpu.make_async_copy(
                rowsv.at[b, pl.ds(j * 128, 128)],
                acc.at[ridx.at[b, j]],
                ssem[b],
            ).wait()

    def compute_ridx(k, b):
        st = k * CH
        for r in range(CH // 16):
            g = lax.iota(jnp.int32, 16) + (st + 16 * r)
            valid = (g >= sc_lo) & (g < sc_hi)
            rel = jnp.where(valid, segv[b, pl.ds(16 * r, 16)] - seg_base, TRASH)
            ridx[b, r // 8, pl.ds(16 * (r % 8), 16)] = rel

    def mul(b):
        def mul_body(i, carry2):
            for u in range(16):
                t = i * 16 + u
                rowsv[b, t, :] = tbv[b, pl.ds(t * NB_C, NB_C)] * rowsv[b, t, :]
            return carry2

        lax.fori_loop(0, CH // 16, mul_body, 0)

    trash_row = jnp.full((16,), TRASH, jnp.int32)

    @pl.when(nch > 0)
    def _run():
        for b in range(2):
            for j in range(NJ):
                for r in range(8):
                    ridx[b, j, pl.ds(16 * r, 16)] = trash_row
        issue_scatters(0)
        issue_scatters(1)
        issue_linear(ka, 0)

        npairs = (nch + 1) // 2

        def pair_body(i2, carry):
            kA = ka + 2 * i2
            kB = kA + 1
            vB = kB < kb
            wait_linear(0)
            drain_scatters(0)
            hsA = issue_gathers(0)

            @pl.when(vB)
            def _():
                issue_linear(kB, 1)

            compute_ridx(kA, 0)
            for h in hsA:
                h.wait()
            mul(0)
            issue_scatters(0)

            @pl.when(kA + 2 < kb)
            def _():
                issue_linear(kA + 2, 0)

            @pl.when(vB)
            def _():
                wait_linear(1)
                drain_scatters(1)
                hsB = issue_gathers(1)
                compute_ridx(kB, 1)
                for h in hsB:
                    h.wait()
                mul(1)
                issue_scatters(1)

            return carry

        lax.fori_loop(0, npairs, pair_body, 0)
        drain_scatters(0)
        drain_scatters(1)

    plsc.subcore_barrier()

    out_base = c * SEGS_PER_SC + s * (SEGS_PER_SC // NS)
    pltpu.sync_copy(
        acc.at[pl.ds(s * (SEGS_PER_SC // NS), SEGS_PER_SC // NS)],
        out_hbm.at[pl.ds(out_base, SEGS_PER_SC // NS)],
    )


def _triples_sc(tb_flat, segment_ids, lg_dst, edge_atoms, split_arr):
    mesh = plsc.VectorSubcoreMesh(
        core_axis_name="c", subcore_axis_name="s", num_cores=NC, num_subcores=NS
    )
    kern = pl.kernel(
        _triples_body,
        out_type=jax.ShapeDtypeStruct((N_EDGES_C, NB_C), jnp.float32),
        mesh=mesh,
        compiler_params=pltpu.CompilerParams(use_tc_tiling_on_sc=False),
        scratch_types=[
            pltpu.VMEM((2, CH), jnp.int32),
            pltpu.VMEM((2, CH), jnp.int32),
            pltpu.VMEM((2, CH * NB_C), jnp.float32),
            pltpu.VMEM((2, CH, NB_C), jnp.float32),
            pltpu.VMEM((2, NJ, 128), jnp.int32),
            pltpu.VMEM((313, NB_C), jnp.float32),
            pltpu.VMEM((16,), jnp.int32),
            pltpu.VMEM_SHARED((ACC_ROWS, NB_C), jnp.float32),
            pltpu.SemaphoreType.DMA,
            pltpu.SemaphoreType.DMA,
            pltpu.SemaphoreType.DMA,
            pltpu.SemaphoreType.DMA,
            pltpu.SemaphoreType.DMA,
            pltpu.SemaphoreType.DMA,
        ],
    )
    return kern(tb_flat, segment_ids, lg_dst, edge_atoms, split_arr)


def _mlp_body(nb_ref, efT_ref, wb_ref, bbT_ref, wg_ref, bgT_ref, oT_ref):
    nb = nb_ref[...]
    dims = (((0,), (1,)), ((), ()))
    uT = lax.dot_general(wb_ref[...], nb, dims,
                         preferred_element_type=jnp.float32) + bbT_ref[...]
    gT = jax.nn.sigmoid(
        lax.dot_general(wg_ref[...], nb, dims,
                        preferred_element_type=jnp.float32) + bgT_ref[...]
    )
    oT_ref[...] = efT_ref[...] + uT * gT


def _mlp_tc(new_bonds, edge_feat, W_bond, b_bond, W_gate, b_gate):
    efT = edge_feat.T
    BE = 3200
    grid = (N_EDGES_C // BE,)
    outT = pl.pallas_call(
        _mlp_body,
        grid=grid,
        in_specs=[
            pl.BlockSpec((BE, NB_C), lambda i: (i, 0)),
            pl.BlockSpec((D_EDGE_C, BE), lambda i: (0, i)),
            pl.BlockSpec((NB_C, D_EDGE_C), lambda i: (0, 0)),
            pl.BlockSpec((D_EDGE_C, 1), lambda i: (0, 0)),
            pl.BlockSpec((NB_C, D_EDGE_C), lambda i: (0, 0)),
            pl.BlockSpec((D_EDGE_C, 1), lambda i: (0, 0)),
        ],
        out_specs=pl.BlockSpec((D_EDGE_C, BE), lambda i: (0, i)),
        out_shape=jax.ShapeDtypeStruct((D_EDGE_C, N_EDGES_C), jnp.float32),
    )(new_bonds, efT, W_bond, b_bond.reshape(D_EDGE_C, 1),
      W_gate, b_gate.reshape(D_EDGE_C, 1))
    return outT.T


def kernel(node_feat, edge_feat, three_basis, three_cutoff, edge_dst, lg_src,
           lg_dst, segment_ids, W_atom, b_atom, W_bond, b_bond, W_gate, b_gate):
    del three_cutoff, lg_src
    edge_dst = edge_dst.astype(jnp.int32)
    lg_dst = lg_dst.astype(jnp.int32)
    segment_ids = segment_ids.astype(jnp.int32)

    atoms = _atoms_tc(node_feat, W_atom, b_atom)
    edge_atoms = _edge_atoms_sc(atoms, edge_dst)

    split = jnp.sum(
        (segment_ids < SEGS_PER_SC).astype(jnp.int32), dtype=jnp.int32
    )
    split_arr = jnp.full((16,), split, dtype=jnp.int32)

    tb_flat = three_basis.reshape(-1)
    new_bonds = _triples_sc(tb_flat, segment_ids, lg_dst, edge_atoms, split_arr)
    return _mlp_tc(new_bonds, edge_feat, W_bond, b_bond, W_gate, b_gate)


# speedup vs baseline: 41.2842x; 1.0668x over previous
import functools

import jax
import jax.numpy as jnp
from jax import lax
from jax.experimental import pallas as pl
from jax.experimental.pallas import tpu as pltpu
from jax.experimental.pallas import tpu_sc as plsc

N_NODES_C = 10000
N_EDGES_C = 160000
N_TRIPLES_C = 1280000
D_NODE_C = 128
D_EDGE_C = 64
NB_C = 16

NC = 2
NS = 16

CH = 512
KTOT = N_TRIPLES_C // CH
SEGS_PER_SC = N_EDGES_C // NC
ACC_ROWS = SEGS_PER_SC + 128
TRASH = SEGS_PER_SC
NJ = CH // 128


def _atoms_body(x_ref, w_ref, b_ref, o_ref):
    x = x_ref[...]
    o_ref[...] = jax.nn.sigmoid(
        jnp.dot(x, w_ref[...], preferred_element_type=jnp.float32) + b_ref[...]
    )


def _atoms_tc(node_feat, W_atom, b_atom):
    BN = 2000
    grid = (N_NODES_C // BN,)
    return pl.pallas_call(
        _atoms_body,
        grid=grid,
        in_specs=[
            pl.BlockSpec((BN, D_NODE_C), lambda i: (i, 0)),
            pl.BlockSpec((D_NODE_C, NB_C), lambda i: (0, 0)),
            pl.BlockSpec((1, NB_C), lambda i: (0, 0)),
        ],
        out_specs=pl.BlockSpec((BN, NB_C), lambda i: (i, 0)),
        out_shape=jax.ShapeDtypeStruct((N_NODES_C, NB_C), jnp.float32),
    )(node_feat, W_atom, b_atom.reshape(1, NB_C))


_B_CHUNK = 128
_B_PER_W = 39
_B_EDGES_W = _B_CHUNK * _B_PER_W
_B_TAIL = N_EDGES_C - 32 * _B_EDGES_W


def _edge_atoms_body(atoms_hbm, ed_hbm, out_hbm, idxv, rowsv, sem):
    c = lax.axis_index("c")
    s = lax.axis_index("s")
    w = s * NC + c
    base_e = w * _B_EDGES_W
    pltpu.sync_copy(ed_hbm.at[pl.ds(base_e, _B_EDGES_W)], idxv)
    for wave in range(5):
        nj = 8 if wave < 4 else 7
        handles = []
        for j in range(nj):
            k = wave * 8 + j
            handles.append(
                pltpu.async_copy(
                    atoms_hbm.at[idxv.at[pl.ds(k * _B_CHUNK, _B_CHUNK)]],
                    rowsv.at[pl.ds(j * _B_CHUNK, _B_CHUNK)],
                    sem,
                )
            )
        for h in handles:
            h.wait()
        pltpu.sync_copy(
            rowsv.at[pl.ds(0, nj * _B_CHUNK)],
            out_hbm.at[pl.ds(base_e + wave * 8 * _B_CHUNK, nj * _B_CHUNK)],
        )

    @pl.when(w < 2)
    def _tail():
        tb = 32 * _B_EDGES_W + w * _B_CHUNK
        pltpu.sync_copy(ed_hbm.at[pl.ds(tb, _B_CHUNK)], idxv.at[pl.ds(0, _B_CHUNK)])
        pltpu.async_copy(
            atoms_hbm.at[idxv.at[pl.ds(0, _B_CHUNK)]],
            rowsv.at[pl.ds(0, _B_CHUNK)],
            sem,
        ).wait()
        pltpu.sync_copy(rowsv.at[pl.ds(0, _B_CHUNK)], out_hbm.at[pl.ds(tb, _B_CHUNK)])


def _edge_atoms_sc(atoms, edge_dst):
    mesh = plsc.VectorSubcoreMesh(
        core_axis_name="c", subcore_axis_name="s", num_cores=NC, num_subcores=NS
    )
    kern = pl.kernel(
        _edge_atoms_body,
        out_type=jax.ShapeDtypeStruct((N_EDGES_C, NB_C), jnp.float32),
        mesh=mesh,
        compiler_params=pltpu.CompilerParams(use_tc_tiling_on_sc=False),
        scratch_types=[
            pltpu.VMEM((_B_EDGES_W,), jnp.int32),
            pltpu.VMEM((8 * _B_CHUNK, NB_C), jnp.float32),
            pltpu.SemaphoreType.DMA,
        ],
    )
    return kern(atoms, edge_dst)


def _triples_body(tb_hbm, seg_hbm, lg_hbm, ea_hbm, split_hbm, out_hbm,
                  segv, lgv, tbv, rowsv, ridx, zbuf, spl_v,
                  acc, lsem0, lsem1, gsem0, gsem1, ssem0, ssem1):
    c = lax.axis_index("c")
    s = lax.axis_index("s")
    lsem = (lsem0, lsem1)
    gsem = (gsem0, gsem1)
    ssem = (ssem0, ssem1)

    pltpu.sync_copy(split_hbm, spl_v)
    split = spl_v[...][0]

    zrow = jnp.zeros((NB_C,), jnp.float32)

    def zbody(i, carry):
        zbuf[i, :] = zrow
        return carry

    lax.fori_loop(0, 313, zbody, 0)
    for q in range(16):
        pltpu.sync_copy(zbuf, acc.at[pl.ds(s * 5008 + q * 313, 313)])
    plsc.subcore_barrier()

    sc_lo = jnp.where(c == 0, 0, split)
    sc_hi = jnp.where(c == 0, split, N_TRIPLES_C)
    kc_lo = sc_lo // CH
    kc_hi = (sc_hi + (CH - 1)) // CH
    n = kc_hi - kc_lo
    ka = kc_lo + (n * s) // NS
    kb = kc_lo + (n * (s + 1)) // NS
    nch = kb - ka
    seg_base = c * SEGS_PER_SC

    def issue_linear(k, b):
        st = k * CH
        pltpu.async_copy(seg_hbm.at[pl.ds(st, CH)], segv.at[b], lsem[b])
        pltpu.async_copy(lg_hbm.at[pl.ds(st, CH)], lgv.at[b], lsem[b])
        pltpu.async_copy(tb_hbm.at[pl.ds(st * NB_C, CH * NB_C)], tbv.at[b], lsem[b])

    def wait_linear(b):
        pltpu.make_async_copy(seg_hbm.at[pl.ds(0, CH)], segv.at[b], lsem[b]).wait()
        pltpu.make_async_copy(lg_hbm.at[pl.ds(0, CH)], lgv.at[b], lsem[b]).wait()
        pltpu.make_async_copy(tb_hbm.at[pl.ds(0, CH * NB_C)], tbv.at[b], lsem[b]).wait()

    def issue_gathers(b):
        for j in range(NJ):
            pltpu.async_copy(
                ea_hbm.at[lgv.at[b, pl.ds(j * 128, 128)]],
                rowsv.at[b, pl.ds(j * 128, 128)],
                gsem[b],
            )

    def drain_gathers(b):
        for j in range(NJ):
            pltpu.make_async_copy(
                ea_hbm.at[lgv.at[b, pl.ds(j * 128, 128)]],
                rowsv.at[b, pl.ds(j * 128, 128)],
                gsem[b],
            ).wait()

    def issue_scatters(b):
        for j in range(NJ):
            pltpu.async_copy(
                rowsv.at[b, pl.ds(j * 128, 128)],
                acc.at[ridx.at[b, j]],
                ssem[b],
                add=True,
            )

    def drain_scatters(b):
        for j in range(NJ):
            pltpu.make_async_copy(
                rowsv.at[b, pl.ds(j * 128, 128)],
                acc.at[ridx.at[b, j]],
                ssem[b],
            ).wait()

    def compute_ridx(k, b):
        st = k * CH
        for r in range(CH // 16):
            g = lax.iota(jnp.int32, 16) + (st + 16 * r)
            valid = (g >= sc_lo) & (g < sc_hi)
            rel = jnp.where(valid, segv[b, pl.ds(16 * r, 16)] - seg_base, TRASH)
            ridx[b, r // 8, pl.ds(16 * (r % 8), 16)] = rel

    def mul(b):
        def mul_body(i, carry2):
            for u in range(16):
                t = i * 16 + u
                rowsv[b, t, :] = tbv[b, pl.ds(t * NB_C, NB_C)] * rowsv[b, t, :]
            return carry2

        lax.fori_loop(0, CH // 16, mul_body, 0)

    @pl.when(nch > 0)
    def _run():
        issue_linear(ka, 0)

        @pl.when(ka + 1 < kb)
        def _():
            issue_linear(ka + 1, 1)

        wait_linear(0)
        issue_gathers(0)

        @pl.when(ka + 1 < kb)
        def _():
            wait_linear(1)
            issue_gathers(1)

        npairs = (nch + 1) // 2

        def pair_body(i2, carry):
            kA = ka + 2 * i2
            kB = kA + 1
            vB = kB < kb
            compute_ridx(kA, 0)
            drain_gathers(0)
            mul(0)
            issue_scatters(0)

            @pl.when(kA + 2 < kb)
            def _():
                issue_linear(kA + 2, 0)

            @pl.when(vB)
            def _():
                compute_ridx(kB, 1)
                drain_gathers(1)
                mul(1)
                issue_scatters(1)

                @pl.when(kB + 2 < kb)
                def _():
                    issue_linear(kB + 2, 1)

            drain_scatters(0)

            @pl.when(kA + 2 < kb)
            def _():
                wait_linear(0)
                issue_gathers(0)

            @pl.when(vB)
            def _():
                drain_scatters(1)

                @pl.when(kB + 2 < kb)
                def _():
                    wait_linear(1)
                    issue_gathers(1)

            return carry

        lax.fori_loop(0, npairs, pair_body, 0)

    plsc.subcore_barrier()

    out_base = c * SEGS_PER_SC + s * (SEGS_PER_SC // NS)
    pltpu.sync_copy(
        acc.at[pl.ds(s * (SEGS_PER_SC // NS), SEGS_PER_SC // NS)],
        out_hbm.at[pl.ds(out_base, SEGS_PER_SC // NS)],
    )


def _triples_sc(tb_flat, segment_ids, lg_dst, edge_atoms, split_arr):
    mesh = plsc.VectorSubcoreMesh(
        core_axis_name="c", subcore_axis_name="s", num_cores=NC, num_subcores=NS
    )
    kern = pl.kernel(
        _triples_body,
        out_type=jax.ShapeDtypeStruct((N_EDGES_C, NB_C), jnp.float32),
        mesh=mesh,
        compiler_params=pltpu.CompilerParams(use_tc_tiling_on_sc=False),
        scratch_types=[
            pltpu.VMEM((2, CH), jnp.int32),
            pltpu.VMEM((2, CH), jnp.int32),
            pltpu.VMEM((2, CH * NB_C), jnp.float32),
            pltpu.VMEM((2, CH, NB_C), jnp.float32),
            pltpu.VMEM((2, NJ, 128), jnp.int32),
            pltpu.VMEM((313, NB_C), jnp.float32),
            pltpu.VMEM((16,), jnp.int32),
            pltpu.VMEM_SHARED((ACC_ROWS, NB_C), jnp.float32),
            pltpu.SemaphoreType.DMA,
            pltpu.SemaphoreType.DMA,
            pltpu.SemaphoreType.DMA,
            pltpu.SemaphoreType.DMA,
            pltpu.SemaphoreType.DMA,
            pltpu.SemaphoreType.DMA,
        ],
    )
    return kern(tb_flat, segment_ids, lg_dst, edge_atoms, split_arr)


def _mlp_body(nb_ref, efT_ref, wb_ref, bbT_ref, wg_ref, bgT_ref, oT_ref):
    nb = nb_ref[...]
    dims = (((0,), (1,)), ((), ()))
    uT = lax.dot_general(wb_ref[...], nb, dims,
                         preferred_element_type=jnp.float32) + bbT_ref[...]
    gT = jax.nn.sigmoid(
        lax.dot_general(wg_ref[...], nb, dims,
                        preferred_element_type=jnp.float32) + bgT_ref[...]
    )
    oT_ref[...] = efT_ref[...] + uT * gT


def _mlp_tc(new_bonds, edge_feat, W_bond, b_bond, W_gate, b_gate):
    efT = edge_feat.T
    BE = 3200
    grid = (N_EDGES_C // BE,)
    outT = pl.pallas_call(
        _mlp_body,
        grid=grid,
        in_specs=[
            pl.BlockSpec((BE, NB_C), lambda i: (i, 0)),
            pl.BlockSpec((D_EDGE_C, BE), lambda i: (0, i)),
            pl.BlockSpec((NB_C, D_EDGE_C), lambda i: (0, 0)),
            pl.BlockSpec((D_EDGE_C, 1), lambda i: (0, 0)),
            pl.BlockSpec((NB_C, D_EDGE_C), lambda i: (0, 0)),
            pl.BlockSpec((D_EDGE_C, 1), lambda i: (0, 0)),
        ],
        out_specs=pl.BlockSpec((D_EDGE_C, BE), lambda i: (0, i)),
        out_shape=jax.ShapeDtypeStruct((D_EDGE_C, N_EDGES_C), jnp.float32),
    )(new_bonds, efT, W_bond, b_bond.reshape(D_EDGE_C, 1),
      W_gate, b_gate.reshape(D_EDGE_C, 1))
    return outT.T


def kernel(node_feat, edge_feat, three_basis, three_cutoff, edge_dst, lg_src,
           lg_dst, segment_ids, W_atom, b_atom, W_bond, b_bond, W_gate, b_gate):
    del three_cutoff, lg_src
    edge_dst = edge_dst.astype(jnp.int32)
    lg_dst = lg_dst.astype(jnp.int32)
    segment_ids = segment_ids.astype(jnp.int32)

    atoms = _atoms_tc(node_feat, W_atom, b_atom)
    edge_atoms = _edge_atoms_sc(atoms, edge_dst)

    split = jnp.sum(
        (segment_ids < SEGS_PER_SC).astype(jnp.int32), dtype=jnp.int32
    )
    split_arr = jnp.full((16,), split, dtype=jnp.int32)

    tb_flat = three_basis.reshape(-1)
    new_bonds = _triples_sc(tb_flat, segment_ids, lg_dst, edge_atoms, split_arr)
    return _mlp_tc(new_bonds, edge_feat, W_bond, b_bond, W_gate, b_gate)
